# baseline (speedup 1.0000x reference)
import functools

import jax
import jax.numpy as jnp
from jax import lax
from jax.experimental import pallas as pl
from jax.experimental.pallas import tpu as pltpu
from jax.experimental.pallas import tpu_sc as plsc

_B = 16384
_D = 32
_NC = 2
_NS = 16
_NW = _NC * _NS
_BPW = _B // _NW


def _gather_kernel(idx_hbm, tabT_hbm, out_hbm, ix_v, rows_v, sem):
    wid = lax.axis_index("s") * _NC + lax.axis_index("c")
    base = wid * _BPW
    pltpu.sync_copy(idx_hbm.at[wid], ix_v)

    def body(d):
        for c in range(_BPW // 128):
            pltpu.async_copy(
                tabT_hbm.at[d].at[ix_v.at[pl.ds(c * 128, 128)]],
                rows_v.at[d, pl.ds(c * 128, 128)], sem)
    pl.loop(0, _D)(body)
    pltpu.make_async_copy(tabT_hbm.at[pl.ds(0, _D), pl.ds(0, _BPW)],
                          rows_v, sem).wait()
    pltpu.sync_copy(rows_v, out_hbm.at[:, pl.ds(base, _BPW)])


@jax.jit
def _gather_t(idx, tabT):
    mesh = plsc.VectorSubcoreMesh(core_axis_name="c", subcore_axis_name="s")
    return pl.kernel(
        _gather_kernel,
        mesh=mesh,
        compiler_params=pltpu.CompilerParams(use_tc_tiling_on_sc=False),
        out_type=jax.ShapeDtypeStruct((_D, _B), jnp.float32),
        scratch_types=[
            pltpu.VMEM((_BPW,), jnp.int32),
            pltpu.VMEM((_D, _BPW), jnp.float32),
            pltpu.SemaphoreType.DMA,
        ],
    )(idx, tabT)


_BLK = 2048


def _mlp_kernel(ue_ref, ie_ref, w1at_ref, w1bt_ref, b1_ref, w2t_ref, b2_ref,
                w3_ref, b3_ref, out_ref):
    x = (jnp.dot(w1at_ref[...], ue_ref[...], preferred_element_type=jnp.float32)
         + jnp.dot(w1bt_ref[...], ie_ref[...], preferred_element_type=jnp.float32)
         + b1_ref[...])
    h1 = jnp.maximum(x, 0.0)
    h2 = jnp.maximum(
        jnp.dot(w2t_ref[...], h1, preferred_element_type=jnp.float32)
        + b2_ref[...], 0.0)
    logit = jnp.dot(w3_ref[...], h2, preferred_element_type=jnp.float32)
    out_ref[...] = jax.nn.sigmoid(logit + b3_ref[0, 0])[0]


@jax.jit
def _mlp(ueT, ieT, w1at, w1bt, b1, w2t, b2, w3, b3):
    grid = (_B // _BLK,)
    full = lambda i: (0, 0)
    return pl.pallas_call(
        _mlp_kernel,
        grid=grid,
        in_specs=[
            pl.BlockSpec((_D, _BLK), lambda i: (0, i)),
            pl.BlockSpec((_D, _BLK), lambda i: (0, i)),
            pl.BlockSpec((128, _D), full),
            pl.BlockSpec((128, _D), full),
            pl.BlockSpec((128, 1), full),
            pl.BlockSpec((64, 128), full),
            pl.BlockSpec((64, 1), full),
            pl.BlockSpec((1, 64), full),
            pl.BlockSpec((1, 1), full),
        ],
        out_specs=pl.BlockSpec((_BLK,), lambda i: (i,)),
        out_shape=jax.ShapeDtypeStruct((_B,), jnp.float32),
    )(ueT, ieT, w1at, w1bt, b1, w2t, b2, w3, b3)


def kernel(users, items, user_table, item_table, W1, b1, W2, b2, W3, b3):
    ieT = _gather_t(items.reshape(_NW, _BPW), item_table.T)
    ueT = _gather_t(users.reshape(_NW, _BPW), user_table.T)
    return _mlp(ueT, ieT, W1[:_D].T, W1[_D:].T, b1.reshape(128, 1),
                W2.T, b2.reshape(64, 1), W3.reshape(1, 64), b3.reshape(1, 1))


# speedup vs baseline: 8.1070x; 8.1070x over previous
import functools

import jax
import jax.numpy as jnp
from jax import lax
from jax.experimental import pallas as pl
from jax.experimental.pallas import tpu as pltpu
from jax.experimental.pallas import tpu_sc as plsc

_B = 16384
_D = 32
_NC = 2
_NS = 16
_NW = _NC * _NS
_BPW = _B // _NW


def _gather_kernel(uidx_hbm, iidx_hbm, utab_hbm, itab_hbm, ue_hbm, ie_hbm,
                   uix_v, iix_v, rows_v, sem):
    wid = lax.axis_index("s") * _NC + lax.axis_index("c")
    base = wid * _BPW
    pltpu.sync_copy(uidx_hbm.at[wid], uix_v)
    pltpu.sync_copy(iidx_hbm.at[wid], iix_v)

    def stage(idx_v, tab_hbm, out_hbm):
        def grp(g):
            vec = idx_v[pl.ds(g * 16, 16)]
            for k in range(16):
                r = vec[k]
                pltpu.async_copy(tab_hbm.at[pl.ds(r, 1)],
                                 rows_v.at[pl.ds(g * 16 + k, 1)], sem)
        pl.loop(0, _BPW // 16)(grp)
        pltpu.make_async_copy(tab_hbm.at[pl.ds(0, _BPW)], rows_v, sem).wait()
        pltpu.sync_copy(rows_v, out_hbm.at[pl.ds(base, _BPW)])

    stage(uix_v, utab_hbm, ue_hbm)
    stage(iix_v, itab_hbm, ie_hbm)


@jax.jit
def _gather(uidx, iidx, user_table, item_table):
    mesh = plsc.VectorSubcoreMesh(core_axis_name="c", subcore_axis_name="s")
    return pl.kernel(
        _gather_kernel,
        mesh=mesh,
        compiler_params=pltpu.CompilerParams(use_tc_tiling_on_sc=True),
        out_type=(
            jax.ShapeDtypeStruct((_B, _D), jnp.float32),
            jax.ShapeDtypeStruct((_B, _D), jnp.float32),
        ),
        scratch_types=[
            pltpu.VMEM((_BPW,), jnp.int32),
            pltpu.VMEM((_BPW,), jnp.int32),
            pltpu.VMEM((_BPW, _D), jnp.float32),
            pltpu.SemaphoreType.DMA,
        ],
    )(uidx, iidx, user_table, item_table)


_BLK = 2048


def _mlp_kernel(ue_ref, ie_ref, w1a_ref, w1b_ref, b1_ref, w2_ref, b2_ref,
                w3_ref, b3_ref, out_ref):
    x = (jnp.dot(ue_ref[...], w1a_ref[...], preferred_element_type=jnp.float32)
         + jnp.dot(ie_ref[...], w1b_ref[...], preferred_element_type=jnp.float32)
         + b1_ref[...])
    h1 = jnp.maximum(x, 0.0)
    h2 = jnp.maximum(
        jnp.dot(h1, w2_ref[...], preferred_element_type=jnp.float32)
        + b2_ref[...], 0.0)
    logit = jnp.sum(h2 * w3_ref[...], axis=1) + b3_ref[0, 0]
    out_ref[...] = jax.nn.sigmoid(logit)


@jax.jit
def _mlp(ue, ie, w1a, w1b, b1, w2, b2, w3, b3):
    grid = (_B // _BLK,)
    full = lambda i: (0, 0)
    return pl.pallas_call(
        _mlp_kernel,
        grid=grid,
        in_specs=[
            pl.BlockSpec((_BLK, _D), lambda i: (i, 0)),
            pl.BlockSpec((_BLK, _D), lambda i: (i, 0)),
            pl.BlockSpec((_D, 128), full),
            pl.BlockSpec((_D, 128), full),
            pl.BlockSpec((1, 128), full),
            pl.BlockSpec((128, 64), full),
            pl.BlockSpec((1, 64), full),
            pl.BlockSpec((1, 64), full),
            pl.BlockSpec((1, 1), full),
        ],
        out_specs=pl.BlockSpec((_BLK,), lambda i: (i,)),
        out_shape=jax.ShapeDtypeStruct((_B,), jnp.float32),
    )(ue, ie, w1a, w1b, b1, w2, b2, w3, b3)


def kernel(users, items, user_table, item_table, W1, b1, W2, b2, W3, b3):
    uidx = users.reshape(_NW, _BPW)
    iidx = items.reshape(_NW, _BPW)
    ue, ie = _gather(uidx, iidx, user_table, item_table)
    return _mlp(ue, ie, W1[:_D], W1[_D:], b1.reshape(1, 128),
                W2, b2.reshape(1, 64), W3.reshape(1, 64), b3.reshape(1, 1))
